# baseline (speedup 1.0000x reference)
import functools

import jax
import jax.numpy as jnp
from jax import lax
from jax.experimental import pallas as pl
from jax.experimental.pallas import tpu as pltpu
from jax.experimental.pallas import tpu_sc as plsc

_NS = [32768, 8192, 2048, 512, 128]
_CS = [32, 64, 128, 256, 512]
_NWORKERS = 32


def _knn_body(q_ref, stc_ref, idx_ref, *, tile, n_src):
    q = q_ref[...]
    n_chunks = n_src // 128
    chunk_bits = (n_chunks - 1).bit_length()
    keep_mask = jnp.int32(-(1 << chunk_bits)) if chunk_bits else jnp.int32(-1)
    chunk_mask = jnp.int32(n_chunks - 1)
    big = jnp.full((tile, 128), 3.0e38, jnp.float32)

    def chunk(c, carry):
        m1, m2, m3 = carry
        st = stc_ref[c]
        d2 = jnp.zeros((tile, 128), jnp.float32)
        for d in range(3):
            diff = q[:, d:d + 1] - st[d:d + 1, :]
            d2 = d2 + diff * diff
        bits = lax.bitcast_convert_type(d2, jnp.int32)
        key = lax.bitcast_convert_type((bits & keep_mask) | c, jnp.float32)
        m1n = jnp.minimum(m1, key)
        h1 = jnp.maximum(m1, key)
        m2n = jnp.minimum(m2, h1)
        h2 = jnp.maximum(m2, h1)
        m3n = jnp.minimum(m3, h2)
        return m1n, m2n, m3n

    m1, m2, m3 = lax.fori_loop(0, n_chunks, chunk, (big, big, big))

    inf = jnp.float32(jnp.inf)
    lane = lax.broadcasted_iota(jnp.int32, (tile, 128), 1)

    def col_of(t, l):
        return (lax.bitcast_convert_type(t, jnp.int32) & chunk_mask) * 128 + l

    t1 = jnp.min(m1, axis=1, keepdims=True)
    hit1 = m1 == t1
    l1 = jnp.min(jnp.where(hit1, lane, 128), axis=1, keepdims=True)
    m1a = jnp.where(hit1 & (lane == l1), inf, m1)

    t2 = jnp.minimum(jnp.min(m1a, axis=1, keepdims=True),
                     jnp.min(m2, axis=1, keepdims=True))
    hit2a = m1a == t2
    hit2b = m2 == t2
    l2 = jnp.min(jnp.where(hit2a | hit2b, lane, 128), axis=1, keepdims=True)
    sel2 = lane == l2
    m1b = jnp.where(hit2a & sel2, inf, m1a)
    m2b = jnp.where(hit2b & sel2, inf, m2)

    t3 = jnp.minimum(jnp.minimum(jnp.min(m1b, axis=1, keepdims=True),
                                 jnp.min(m2b, axis=1, keepdims=True)),
                     jnp.min(m3, axis=1, keepdims=True))
    l3 = jnp.min(jnp.where((m1b == t3) | (m2b == t3) | (m3 == t3), lane, 128),
                 axis=1, keepdims=True)

    ids = [col_of(t1, l1), col_of(t2, l2), col_of(t3, l3)]
    idx_ref[...] = jnp.concatenate(ids + [jnp.zeros((tile, 5), jnp.int32)],
                                   axis=1)


def _knn(p_dst_pad, p_src_chunks, n_dst, n_src, tile):
    grid = n_dst // tile
    n_chunks = n_src // 128
    return pl.pallas_call(
        functools.partial(_knn_body, tile=tile, n_src=n_src),
        grid=(grid,),
        in_specs=[pl.BlockSpec((tile, 8), lambda i: (i, 0)),
                  pl.BlockSpec((n_chunks, 8, 128), lambda i: (0, 0, 0))],
        out_specs=pl.BlockSpec((tile, 8), lambda i: (i, 0)),
        out_shape=jax.ShapeDtypeStruct((n_dst, 8), jnp.int32),
    )(p_dst_pad, p_src_chunks)


def _sc_gather(f_src, idx_flat, n_dst, c):
    rows_n = n_dst // _NWORKERS
    chunk = min(rows_n, 65536 // c)
    n_chunk = rows_n // chunk
    sub = min(chunk, 128)
    n_sub = chunk // sub
    mesh = plsc.VectorSubcoreMesh(core_axis_name="c", subcore_axis_name="s")

    @functools.partial(
        pl.kernel, mesh=mesh,
        out_type=jax.ShapeDtypeStruct((3, n_dst, c), jnp.float32),
        scratch_types=[pltpu.VMEM((chunk,), jnp.int32),
                       pltpu.VMEM((chunk, c), jnp.float32),
                       pltpu.SemaphoreType.DMA],
        compiler_params=pltpu.CompilerParams(use_tc_tiling_on_sc=False),
    )
    def gather_kernel(f_hbm, idx_hbm, out_hbm, idx_v, rows_v, sem):
        wid = lax.axis_index("s") * 2 + lax.axis_index("c")
        base = wid * rows_n

        def body(t, carry):
            k = t // n_chunk
            off = base + (t % n_chunk) * chunk
            pltpu.sync_copy(idx_hbm.at[pl.ds(k * n_dst + off, chunk)], idx_v)
            if n_sub == 1:
                copies = [pltpu.async_copy(f_hbm.at[idx_v], rows_v, sem)]
            else:
                copies = [
                    pltpu.async_copy(f_hbm.at[idx_v.at[pl.ds(j * sub, sub)]],
                                     rows_v.at[pl.ds(j * sub, sub)], sem)
                    for j in range(n_sub)
                ]
            for cp in copies:
                cp.wait()
            pltpu.sync_copy(rows_v, out_hbm.at[k, pl.ds(off, chunk)])
            return carry

        lax.fori_loop(0, 3 * n_chunk, body, 0)

    return gather_kernel(f_src, idx_flat)


def _mlp_a_body(f_ref, q_ref, r_ref, wa_ref, wb_ref, b_ref, y_ref, acc_ref,
                *, c_src):
    q = q_ref[...]
    ws = []
    for k in range(3):
        diff = q - r_ref[k][:, c_src:c_src + 8]
        d2 = jnp.sum(diff * diff, axis=1, keepdims=True)
        ws.append(1.0 / jnp.maximum(d2, 1e-16))
    inv = 1.0 / (ws[0] + ws[1] + ws[2])
    up = (ws[0] * inv * r_ref[0] + ws[1] * inv * r_ref[1]
          + ws[2] * inv * r_ref[2])
    y = (jnp.dot(f_ref[...], wa_ref[...], preferred_element_type=jnp.float32)
         + jnp.dot(up, wb_ref[...], preferred_element_type=jnp.float32)
         + b_ref[...])
    y_ref[...] = y
    s0 = jnp.sum(y, axis=0, keepdims=True)
    s1 = jnp.sum(y * y, axis=0, keepdims=True)
    upd = jnp.concatenate([s0, s1, jnp.zeros((6, y.shape[1]), jnp.float32)],
                          axis=0)

    @pl.when(pl.program_id(0) == 0)
    def _():
        acc_ref[...] = upd

    @pl.when(pl.program_id(0) != 0)
    def _():
        acc_ref[...] += upd


def _mlp_a(f_dst, p_dst_pad, rows, wa, wb, b0, c_src, tile):
    n, cf = f_dst.shape
    c_gat = rows.shape[2]
    c1 = wa.shape[1]
    grid = n // tile
    return pl.pallas_call(
        functools.partial(_mlp_a_body, c_src=c_src),
        grid=(grid,),
        in_specs=[pl.BlockSpec((tile, cf), lambda i: (i, 0)),
                  pl.BlockSpec((tile, 8), lambda i: (i, 0)),
                  pl.BlockSpec((3, tile, c_gat), lambda i: (0, i, 0)),
                  pl.BlockSpec((cf, c1), lambda i: (0, 0)),
                  pl.BlockSpec((c_gat, c1), lambda i: (0, 0)),
                  pl.BlockSpec((1, c1), lambda i: (0, 0))],
        out_specs=[pl.BlockSpec((tile, c1), lambda i: (i, 0)),
                   pl.BlockSpec((8, c1), lambda i: (0, 0))],
        out_shape=[jax.ShapeDtypeStruct((n, c1), jnp.float32),
                   jax.ShapeDtypeStruct((8, c1), jnp.float32)],
    )(f_dst, p_dst_pad, rows, wa, wb, b0)


def _mlp_b_body(y_ref, acc_ref, g_ref, be_ref, w_ref, b_ref, y2_ref, acc2_ref,
                *, inv_n):
    mu = acc_ref[0:1, :] * inv_n
    var = acc_ref[1:2, :] * inv_n - mu * mu
    rstd = lax.rsqrt(var + 1e-5)
    x = jnp.maximum((y_ref[...] - mu) * rstd * g_ref[...] + be_ref[...], 0.0)
    y2 = jnp.dot(x, w_ref[...], preferred_element_type=jnp.float32) + b_ref[...]
    y2_ref[...] = y2
    s0 = jnp.sum(y2, axis=0, keepdims=True)
    s1 = jnp.sum(y2 * y2, axis=0, keepdims=True)
    upd = jnp.concatenate([s0, s1, jnp.zeros((6, y2.shape[1]), jnp.float32)],
                          axis=0)

    @pl.when(pl.program_id(0) == 0)
    def _():
        acc2_ref[...] = upd

    @pl.when(pl.program_id(0) != 0)
    def _():
        acc2_ref[...] += upd


def _mlp_b(y1, acc1, g0, be0, w1, b1, tile):
    n, c1 = y1.shape
    c2 = w1.shape[1]
    grid = n // tile
    return pl.pallas_call(
        functools.partial(_mlp_b_body, inv_n=1.0 / n),
        grid=(grid,),
        in_specs=[pl.BlockSpec((tile, c1), lambda i: (i, 0)),
                  pl.BlockSpec((8, c1), lambda i: (0, 0)),
                  pl.BlockSpec((1, c1), lambda i: (0, 0)),
                  pl.BlockSpec((1, c1), lambda i: (0, 0)),
                  pl.BlockSpec((c1, c2), lambda i: (0, 0)),
                  pl.BlockSpec((1, c2), lambda i: (0, 0))],
        out_specs=[pl.BlockSpec((tile, c2), lambda i: (i, 0)),
                   pl.BlockSpec((8, c2), lambda i: (0, 0))],
        out_shape=[jax.ShapeDtypeStruct((n, c2), jnp.float32),
                   jax.ShapeDtypeStruct((8, c2), jnp.float32)],
    )(y1, acc1, g0, be0, w1, b1)


def _mlp_c_body(y_ref, acc_ref, g_ref, be_ref, p_ref, o_ref, *, inv_n, c_pad):
    mu = acc_ref[0:1, :] * inv_n
    var = acc_ref[1:2, :] * inv_n - mu * mu
    rstd = lax.rsqrt(var + 1e-5)
    o = jnp.maximum((y_ref[...] - mu) * rstd * g_ref[...] + be_ref[...], 0.0)
    if c_pad:
        tile, c2 = o.shape
        pieces = [o, p_ref[...]]
        if c_pad > c2 + 8:
            pieces.append(jnp.zeros((tile, c_pad - c2 - 8), jnp.float32))
        o = jnp.concatenate(pieces, axis=1)
    o_ref[...] = o


def _mlp_c(y2, acc2, g1, be1, p_pad, c_pad, tile):
    n, c2 = y2.shape
    grid = n // tile
    c_out = c_pad if c_pad else c2
    return pl.pallas_call(
        functools.partial(_mlp_c_body, inv_n=1.0 / n, c_pad=c_pad),
        grid=(grid,),
        in_specs=[pl.BlockSpec((tile, c2), lambda i: (i, 0)),
                  pl.BlockSpec((8, c2), lambda i: (0, 0)),
                  pl.BlockSpec((1, c2), lambda i: (0, 0)),
                  pl.BlockSpec((1, c2), lambda i: (0, 0)),
                  pl.BlockSpec((tile, 8), lambda i: (i, 0))],
        out_specs=pl.BlockSpec((tile, c_out), lambda i: (i, 0)),
        out_shape=jax.ShapeDtypeStruct((n, c_out), jnp.float32),
    )(y2, acc2, g1, be1, p_pad)


def kernel(p0, f0, b0, p1, f1, b1, p2, f2, b2, p3, f3, b3, p4, f4, b4,
           W0_0, bias0_0, gamma0_0, beta0_0, W0_1, bias0_1, gamma0_1, beta0_1,
           W1_0, bias1_0, gamma1_0, beta1_0, W1_1, bias1_1, gamma1_1, beta1_1,
           W2_0, bias2_0, gamma2_0, beta2_0, W2_1, bias2_1, gamma2_1, beta2_1,
           W3_0, bias3_0, gamma3_0, beta3_0, W3_1, bias3_1, gamma3_1, beta3_1):
    p = [p0, p1, p2, p3, p4]
    f = [f0, f1, f2, f3, f4]
    params = [
        (W0_0, bias0_0, gamma0_0, beta0_0, W0_1, bias0_1, gamma0_1, beta0_1),
        (W1_0, bias1_0, gamma1_0, beta1_0, W1_1, bias1_1, gamma1_1, beta1_1),
        (W2_0, bias2_0, gamma2_0, beta2_0, W2_1, bias2_1, gamma2_1, beta2_1),
        (W3_0, bias3_0, gamma3_0, beta3_0, W3_1, bias3_1, gamma3_1, beta3_1),
    ]
    pad8 = [jnp.pad(x, ((0, 0), (0, 5))) for x in p]
    pchunks = [jnp.transpose(x.reshape(-1, 128, 8), (0, 2, 1)) for x in pad8]

    knn_tile = 64
    mlp_tile = 512
    c_gats = {3: 640, 2: 384, 1: 256, 0: 128}

    idx_flats = {}
    for s in range(3, -1, -1):
        idx8 = _knn(pad8[s], pchunks[s + 1], _NS[s], _NS[s + 1], knn_tile)
        idx_flats[s] = jnp.transpose(idx8[:, :3]).reshape(-1)

    table = jnp.concatenate(
        [f4, pad8[4], jnp.zeros((_NS[4], c_gats[3] - _CS[4] - 8), jnp.float32)],
        axis=1)
    c_src = _CS[4]

    for s in range(3, -1, -1):
        n_dst = _NS[s]
        cf = f[s].shape[1]
        rows = _sc_gather(table, idx_flats[s], n_dst, c_gats[s])
        w0, b0_, g0, be0, w1, b1_, g1, be1 = params[s]
        wa, wb = w0[:cf], w0[cf:]
        wb = jnp.pad(wb, ((0, c_gats[s] - wb.shape[0]), (0, 0)))
        tile = min(mlp_tile, n_dst)
        y1, acc1 = _mlp_a(f[s], pad8[s], rows, wa, wb, b0_.reshape(1, -1),
                          c_src, tile)
        y2, acc2 = _mlp_b(y1, acc1, g0.reshape(1, -1), be0.reshape(1, -1),
                          w1, b1_.reshape(1, -1), tile)
        c_pad = 0 if s == 0 else c_gats[s - 1]
        table = _mlp_c(y2, acc2, g1.reshape(1, -1), be1.reshape(1, -1),
                       pad8[s], c_pad, tile)
        c_src = w1.shape[1]
    return table


# speedup vs baseline: 3.7718x; 3.7718x over previous
import functools

import jax
import jax.numpy as jnp
from jax import lax
from jax.experimental import pallas as pl
from jax.experimental.pallas import tpu as pltpu
from jax.experimental.pallas import tpu_sc as plsc

_NS = [32768, 8192, 2048, 512, 128]
_CS = [32, 64, 128, 256, 512]
_NWORKERS = 32


def _knn_body(q_ref, stc_ref, idx_ref, w_ref, *, tile, n_src):
    g_chunks = n_src // 128
    chunk_bits = (g_chunks - 1).bit_length()
    keep_mask = jnp.int32(-(1 << chunk_bits)) if chunk_bits else jnp.int32(-1)
    chunk_mask = jnp.int32(g_chunks - 1)
    q = q_ref[...]
    stc = stc_ref[...]
    d2 = jnp.zeros((g_chunks, tile, 128), jnp.float32)
    for d in range(3):
        qd = q[:, d:d + 1].reshape(1, tile, 1)
        sd = stc[:, d:d + 1, :].reshape(g_chunks, 1, 128)
        diff = qd - sd
        d2 = d2 + diff * diff
    bits = lax.bitcast_convert_type(d2, jnp.int32)
    gid = lax.broadcasted_iota(jnp.int32, (g_chunks, tile, 128), 0)
    keys = lax.bitcast_convert_type((bits & keep_mask) | gid, jnp.float32)

    inf = jnp.float32(jnp.inf)
    m1 = jnp.min(keys, axis=0)
    k2 = jnp.where(keys == m1[None], inf, keys)
    m2 = jnp.min(k2, axis=0)
    k3 = jnp.where(k2 == m2[None], inf, k2)
    m3 = jnp.min(k3, axis=0)

    lane = lax.broadcasted_iota(jnp.int32, (tile, 128), 1)

    def pick(m, t):
        l = jnp.min(jnp.where(m == t, lane, 128), axis=1, keepdims=True)
        tb = lax.bitcast_convert_type(t, jnp.int32)
        col = (tb & chunk_mask) * 128 + l
        d2v = lax.bitcast_convert_type(tb & keep_mask, jnp.float32)
        return col, d2v

    t1 = jnp.min(m1, axis=1, keepdims=True)
    i1, d1v = pick(m1, t1)
    m1a = jnp.where(m1 == t1, inf, m1)
    t2 = jnp.minimum(jnp.min(m1a, axis=1, keepdims=True),
                     jnp.min(m2, axis=1, keepdims=True))
    i2, d2v_ = pick(jnp.minimum(m1a, m2), t2)
    m1b = jnp.where(m1a == t2, inf, m1a)
    m2b = jnp.where(m2 == t2, inf, m2)
    t3 = jnp.minimum(jnp.minimum(jnp.min(m1b, axis=1, keepdims=True),
                                 jnp.min(m2b, axis=1, keepdims=True)),
                     jnp.min(m3, axis=1, keepdims=True))
    i3, d3v = pick(jnp.minimum(jnp.minimum(m1b, m2b), m3), t3)

    w = [1.0 / jnp.maximum(d, 1e-16) for d in (d1v, d2v_, d3v)]
    wsum = w[0] + w[1] + w[2]
    wn = [x / wsum for x in w]
    zi = jnp.zeros((tile, 5), jnp.int32)
    zf = jnp.zeros((tile, 5), jnp.float32)
    idx_ref[...] = jnp.concatenate([i1, i2, i3, zi], axis=1)
    w_ref[...] = jnp.concatenate(wn + [zf], axis=1)


def _knn(p_dst_pad, p_src_chunks, n_dst, n_src, tile):
    grid = n_dst // tile
    g_chunks = n_src // 128
    return pl.pallas_call(
        functools.partial(_knn_body, tile=tile, n_src=n_src),
        grid=(grid,),
        in_specs=[pl.BlockSpec((tile, 8), lambda i: (i, 0)),
                  pl.BlockSpec((g_chunks, 8, 128), lambda i: (0, 0, 0))],
        out_specs=[pl.BlockSpec((tile, 8), lambda i: (i, 0)),
                   pl.BlockSpec((tile, 8), lambda i: (i, 0))],
        out_shape=[jax.ShapeDtypeStruct((n_dst, 8), jnp.int32),
                   jax.ShapeDtypeStruct((n_dst, 8), jnp.float32)],
    )(p_dst_pad, p_src_chunks)


def _sc_gather(f_src, idx_flat, n_dst, c):
    rows_n = n_dst // _NWORKERS
    chunk = min(rows_n, 65536 // c)
    n_chunk = rows_n // chunk
    sub = min(chunk, 128)
    n_sub = chunk // sub
    mesh = plsc.VectorSubcoreMesh(core_axis_name="c", subcore_axis_name="s")

    @functools.partial(
        pl.kernel, mesh=mesh,
        out_type=jax.ShapeDtypeStruct((3, n_dst, c), jnp.float32),
        scratch_types=[pltpu.VMEM((chunk,), jnp.int32),
                       pltpu.VMEM((chunk, c), jnp.float32),
                       pltpu.SemaphoreType.DMA],
        compiler_params=pltpu.CompilerParams(use_tc_tiling_on_sc=False),
    )
    def gather_kernel(f_hbm, idx_hbm, out_hbm, idx_v, rows_v, sem):
        wid = lax.axis_index("s") * 2 + lax.axis_index("c")
        base = wid * rows_n

        def body(t, carry):
            k = t // n_chunk
            off = base + (t % n_chunk) * chunk
            pltpu.sync_copy(idx_hbm.at[pl.ds(k * n_dst + off, chunk)], idx_v)
            if n_sub == 1:
                copies = [pltpu.async_copy(f_hbm.at[idx_v], rows_v, sem)]
            else:
                copies = [
                    pltpu.async_copy(f_hbm.at[idx_v.at[pl.ds(j * sub, sub)]],
                                     rows_v.at[pl.ds(j * sub, sub)], sem)
                    for j in range(n_sub)
                ]
            for cp in copies:
                cp.wait()
            pltpu.sync_copy(rows_v, out_hbm.at[k, pl.ds(off, chunk)])
            return carry

        lax.fori_loop(0, 3 * n_chunk, body, 0)

    return gather_kernel(f_src, idx_flat)


def _mlp_a_body(f_ref, r_ref, w_ref, wa_ref, wb_ref, b_ref, y_ref, acc_ref):
    up = (w_ref[:, 0:1] * r_ref[0] + w_ref[:, 1:2] * r_ref[1]
          + w_ref[:, 2:3] * r_ref[2])
    y = (jnp.dot(f_ref[...], wa_ref[...], preferred_element_type=jnp.float32)
         + jnp.dot(up, wb_ref[...], preferred_element_type=jnp.float32)
         + b_ref[...])
    y_ref[...] = y
    s0 = jnp.sum(y, axis=0, keepdims=True)
    s1 = jnp.sum(y * y, axis=0, keepdims=True)
    upd = jnp.concatenate([s0, s1, jnp.zeros((6, y.shape[1]), jnp.float32)],
                          axis=0)

    @pl.when(pl.program_id(0) == 0)
    def _():
        acc_ref[...] = upd

    @pl.when(pl.program_id(0) != 0)
    def _():
        acc_ref[...] += upd


def _mlp_a(f_dst, rows, w8, wa, wb, b0, tile):
    n, cf = f_dst.shape
    c_src = rows.shape[2]
    c1 = wa.shape[1]
    grid = n // tile
    return pl.pallas_call(
        _mlp_a_body,
        grid=(grid,),
        in_specs=[pl.BlockSpec((tile, cf), lambda i: (i, 0)),
                  pl.BlockSpec((3, tile, c_src), lambda i: (0, i, 0)),
                  pl.BlockSpec((tile, 8), lambda i: (i, 0)),
                  pl.BlockSpec((cf, c1), lambda i: (0, 0)),
                  pl.BlockSpec((c_src, c1), lambda i: (0, 0)),
                  pl.BlockSpec((1, c1), lambda i: (0, 0))],
        out_specs=[pl.BlockSpec((tile, c1), lambda i: (i, 0)),
                   pl.BlockSpec((8, c1), lambda i: (0, 0))],
        out_shape=[jax.ShapeDtypeStruct((n, c1), jnp.float32),
                   jax.ShapeDtypeStruct((8, c1), jnp.float32)],
    )(f_dst, rows, w8, wa, wb, b0)


def _mlp_b_body(y_ref, acc_ref, g_ref, be_ref, w_ref, b_ref, y2_ref, acc2_ref,
                *, inv_n):
    mu = acc_ref[0:1, :] * inv_n
    var = acc_ref[1:2, :] * inv_n - mu * mu
    rstd = lax.rsqrt(var + 1e-5)
    x = jnp.maximum((y_ref[...] - mu) * rstd * g_ref[...] + be_ref[...], 0.0)
    y2 = jnp.dot(x, w_ref[...], preferred_element_type=jnp.float32) + b_ref[...]
    y2_ref[...] = y2
    s0 = jnp.sum(y2, axis=0, keepdims=True)
    s1 = jnp.sum(y2 * y2, axis=0, keepdims=True)
    upd = jnp.concatenate([s0, s1, jnp.zeros((6, y2.shape[1]), jnp.float32)],
                          axis=0)

    @pl.when(pl.program_id(0) == 0)
    def _():
        acc2_ref[...] = upd

    @pl.when(pl.program_id(0) != 0)
    def _():
        acc2_ref[...] += upd


def _mlp_b(y1, acc1, g0, be0, w1, b1, tile):
    n, c1 = y1.shape
    c2 = w1.shape[1]
    grid = n // tile
    return pl.pallas_call(
        functools.partial(_mlp_b_body, inv_n=1.0 / n),
        grid=(grid,),
        in_specs=[pl.BlockSpec((tile, c1), lambda i: (i, 0)),
                  pl.BlockSpec((8, c1), lambda i: (0, 0)),
                  pl.BlockSpec((1, c1), lambda i: (0, 0)),
                  pl.BlockSpec((1, c1), lambda i: (0, 0)),
                  pl.BlockSpec((c1, c2), lambda i: (0, 0)),
                  pl.BlockSpec((1, c2), lambda i: (0, 0))],
        out_specs=[pl.BlockSpec((tile, c2), lambda i: (i, 0)),
                   pl.BlockSpec((8, c2), lambda i: (0, 0))],
        out_shape=[jax.ShapeDtypeStruct((n, c2), jnp.float32),
                   jax.ShapeDtypeStruct((8, c2), jnp.float32)],
    )(y1, acc1, g0, be0, w1, b1)


def _mlp_c_body(y_ref, acc_ref, g_ref, be_ref, o_ref, *, inv_n):
    mu = acc_ref[0:1, :] * inv_n
    var = acc_ref[1:2, :] * inv_n - mu * mu
    rstd = lax.rsqrt(var + 1e-5)
    o_ref[...] = jnp.maximum(
        (y_ref[...] - mu) * rstd * g_ref[...] + be_ref[...], 0.0)


def _mlp_c(y2, acc2, g1, be1, tile):
    n, c2 = y2.shape
    grid = n // tile
    return pl.pallas_call(
        functools.partial(_mlp_c_body, inv_n=1.0 / n),
        grid=(grid,),
        in_specs=[pl.BlockSpec((tile, c2), lambda i: (i, 0)),
                  pl.BlockSpec((8, c2), lambda i: (0, 0)),
                  pl.BlockSpec((1, c2), lambda i: (0, 0)),
                  pl.BlockSpec((1, c2), lambda i: (0, 0))],
        out_specs=pl.BlockSpec((tile, c2), lambda i: (i, 0)),
        out_shape=jax.ShapeDtypeStruct((n, c2), jnp.float32),
    )(y2, acc2, g1, be1)


def kernel(p0, f0, b0, p1, f1, b1, p2, f2, b2, p3, f3, b3, p4, f4, b4,
           W0_0, bias0_0, gamma0_0, beta0_0, W0_1, bias0_1, gamma0_1, beta0_1,
           W1_0, bias1_0, gamma1_0, beta1_0, W1_1, bias1_1, gamma1_1, beta1_1,
           W2_0, bias2_0, gamma2_0, beta2_0, W2_1, bias2_1, gamma2_1, beta2_1,
           W3_0, bias3_0, gamma3_0, beta3_0, W3_1, bias3_1, gamma3_1, beta3_1):
    p = [p0, p1, p2, p3, p4]
    f = [f0, f1, f2, f3, f4]
    params = [
        (W0_0, bias0_0, gamma0_0, beta0_0, W0_1, bias0_1, gamma0_1, beta0_1),
        (W1_0, bias1_0, gamma1_0, beta1_0, W1_1, bias1_1, gamma1_1, beta1_1),
        (W2_0, bias2_0, gamma2_0, beta2_0, W2_1, bias2_1, gamma2_1, beta2_1),
        (W3_0, bias3_0, gamma3_0, beta3_0, W3_1, bias3_1, gamma3_1, beta3_1),
    ]
    pad8 = [jnp.pad(x, ((0, 0), (0, 5))) for x in p]
    pchunks = [jnp.transpose(x.reshape(-1, 128, 8), (0, 2, 1)) for x in pad8]

    knn_tile = 256
    mlp_tile = 512

    knn = {}
    for s in range(3, -1, -1):
        idx8, w8 = _knn(pad8[s], pchunks[s + 1], _NS[s], _NS[s + 1], knn_tile)
        idx_flat = jnp.transpose(idx8[:, :3]).reshape(-1)
        knn[s] = (idx_flat, w8)

    for s in range(3, -1, -1):
        n_dst = _NS[s]
        cf = f[s].shape[1]
        c_src = f[s + 1].shape[1]
        idx_flat, w8 = knn[s]
        w0, b0_, g0, be0, w1, b1_, g1, be1 = params[s]
        wa, wb = w0[:cf], w0[cf:]
        c_gat = -(-c_src // 128) * 128
        f_src = f[s + 1]
        if c_gat != c_src:
            f_src = jnp.pad(f_src, ((0, 0), (0, c_gat - c_src)))
            wb = jnp.pad(wb, ((0, c_gat - c_src), (0, 0)))
        rows = _sc_gather(f_src, idx_flat, n_dst, c_gat)
        tile = min(mlp_tile, n_dst)
        y1, acc1 = _mlp_a(f[s], rows, w8, wa, wb, b0_.reshape(1, -1), tile)
        y2, acc2 = _mlp_b(y1, acc1, g0.reshape(1, -1), be0.reshape(1, -1),
                          w1, b1_.reshape(1, -1), tile)
        f[s] = _mlp_c(y2, acc2, g1.reshape(1, -1), be1.reshape(1, -1), tile)
    return f[0]
